# baseline (speedup 1.0000x reference)
import functools

import jax
import jax.numpy as jnp
from jax import lax
from jax.experimental import pallas as pl


def _knn_kernel(x_ref, xb_ref, f_ref, wcat_ref, pe_ref, out_ref, ploss_ref,
                *, R, K, J, CH):
    b = pl.program_id(0)
    i = pl.program_id(1)
    N = x_ref.shape[1]
    C = x_ref.shape[2]

    x_all = x_ref[0]
    fm = f_ref[0] > 0.1
    x_crd = jnp.where(fm, 0.0, x_all)
    x_ftr = jnp.where(fm, x_all, 0.0)

    n_f = jnp.float32(N)
    mean_c = jnp.sum(x_crd, axis=0, keepdims=True) / n_f
    mean_f = jnp.sum(x_ftr, axis=0, keepdims=True) / n_f
    var_c = jnp.sum((x_crd - mean_c) ** 2, axis=0, keepdims=True) / n_f
    var_f = jnp.sum((x_ftr - mean_f) ** 2, axis=0, keepdims=True) / n_f
    xn_c = jnp.clip((x_crd - mean_c) / (jnp.sqrt(var_c) + 1e-5), -10.0, 10.0)
    xn_f = jnp.clip((x_ftr - mean_f) / (jnp.sqrt(var_f) + 1e-5), -10.0, 10.0)

    sq_col = jnp.sum(x_crd * x_crd, axis=1, keepdims=True)
    tcol_c = jnp.sum(xn_c, axis=1, keepdims=True)
    tcol_f = jnp.sum(xn_f, axis=1, keepdims=True)

    x_blk_raw = xb_ref[0]
    x_blk = jnp.where(fm, 0.0, x_blk_raw)
    x_blk_f = jnp.where(fm, x_blk_raw, 0.0)
    xnc_blk = jnp.clip((x_blk - mean_c) / (jnp.sqrt(var_c) + 1e-5), -10.0, 10.0)
    xnf_blk = jnp.clip((x_blk_f - mean_f) / (jnp.sqrt(var_f) + 1e-5), -10.0, 10.0)
    tc_blk = jnp.sum(xnc_blk, axis=1, keepdims=True)
    tf_blk = jnp.sum(xnf_blk, axis=1, keepdims=True)

    ones_c = jnp.ones((1, C), jnp.float32)
    sqb_row = lax.dot_general(
        ones_c, x_blk * x_blk, (((1,), (1,)), ((), ())),
        preferred_element_type=jnp.float32,
        precision=lax.Precision.HIGHEST)

    g_t = lax.dot_general(
        x_crd, x_blk, (((1,), (1,)), ((), ())),
        preferred_element_type=jnp.float32,
        precision=lax.Precision.DEFAULT)
    d2t = sq_col + sqb_row - 2.0 * g_t
    dmt = jnp.sqrt(jnp.maximum(d2t, 0.0))

    SL = N // CH
    d3 = dmt.reshape(SL, CH, R)
    tc3 = tcol_c.reshape(SL, CH, 1)
    tf3 = tcol_f.reshape(SL, CH, 1)
    l3 = lax.broadcasted_iota(jnp.int32, (SL, CH, R), 0)
    ci2d = lax.broadcasted_iota(jnp.int32, (CH, R), 0)
    vals, gidxs, tcs1, tfs1 = [], [], [], []
    for _ in range(J):
        v, li, tce, tfe = d3, l3, tc3, tf3
        while v.shape[0] > 1:
            h = v.shape[0] // 2
            le = v[:h] <= v[h:]
            v = jnp.where(le, v[:h], v[h:])
            li = jnp.where(le, li[:h], li[h:])
            tce = jnp.where(le, tce[:h], tce[h:])
            tfe = jnp.where(le, tfe[:h], tfe[h:])
        vals.append(v[0])
        gidxs.append(li[0] * CH + ci2d)
        tcs1.append(tce[0])
        tfs1.append(tfe[0])
        d3 = jnp.where(l3 == li, jnp.inf, d3)

    v_sl = jnp.concatenate(vals, axis=0)
    g_sl = jnp.concatenate(gidxs, axis=0)
    tc_sl = jnp.concatenate(tcs1, axis=0)
    tf_sl = jnp.concatenate(tfs1, axis=0)

    big_g = jnp.int32(N)
    tc_out, tf_out = [], []
    for _ in range(K):
        m2 = jnp.min(v_sl, axis=0, keepdims=True)
        cand2 = jnp.where(v_sl == m2, g_sl, big_g)
        g2 = jnp.min(cand2, axis=0, keepdims=True)
        sel2 = cand2 == g2
        tc_out.append(jnp.sum(jnp.where(sel2, tc_sl, 0.0), axis=0, keepdims=True))
        tf_out.append(jnp.sum(jnp.where(sel2, tf_sl, 0.0), axis=0, keepdims=True))
        v_sl = jnp.where(sel2, jnp.inf, v_sl)

    s_t = jnp.concatenate(tc_out + tf_out, axis=0)
    wcat = wcat_ref[...]
    out = lax.dot_general(
        s_t, wcat, (((0,), (0,)), ((), ())),
        preferred_element_type=jnp.float32,
        precision=lax.Precision.HIGHEST)
    wc_sum = jnp.sum(wcat[:K, :], axis=0, keepdims=True)
    wf_sum = jnp.sum(wcat[K:, :], axis=0, keepdims=True)
    pe = pe_ref[...]
    pe_sum = jnp.sum(pe, axis=0, keepdims=True)
    out_ref[0] = out - tc_blk * wc_sum - tf_blk * wf_sum + pe_sum

    @pl.when((b == 0) & (i == 0))
    def _():
        ploss_ref[...] = jnp.sum(jnp.abs(pe), keepdims=True)


def kernel(x, features, attn_mask, W_crd, W_ftr, pe_crd, pe_ftr):
    del attn_mask
    B, N, C = x.shape
    D, K = W_crd.shape
    R = 256 if N % 256 == 0 else N
    CH = 16 if N % 16 == 0 else 1
    J = 8 if CH > 1 else min(K, N)

    wcat = jnp.concatenate([W_crd.T, W_ftr.T], axis=0)
    pe_cat = jnp.concatenate(
        [pe_crd.reshape(C, D), pe_ftr.reshape(C, D)], axis=0
    )
    f3 = features.reshape(B, 1, C)

    out, ploss = pl.pallas_call(
        functools.partial(_knn_kernel, R=R, K=K, J=J, CH=CH),
        grid=(B, N // R),
        in_specs=[
            pl.BlockSpec((1, N, C), lambda b, i: (b, 0, 0)),
            pl.BlockSpec((1, R, C), lambda b, i: (b, i, 0)),
            pl.BlockSpec((1, 1, C), lambda b, i: (b, 0, 0)),
            pl.BlockSpec((2 * K, D), lambda b, i: (0, 0)),
            pl.BlockSpec((2 * C, D), lambda b, i: (0, 0)),
        ],
        out_specs=[
            pl.BlockSpec((1, R, D), lambda b, i: (b, i, 0)),
            pl.BlockSpec((1, 1), lambda b, i: (0, 0)),
        ],
        out_shape=[
            jax.ShapeDtypeStruct((B, N, D), jnp.float32),
            jax.ShapeDtypeStruct((1, 1), jnp.float32),
        ],
    )(x, x, f3, wcat, pe_cat)
    return out, ploss.reshape(())


# speedup vs baseline: 1.1731x; 1.1731x over previous
import functools

import jax
import jax.numpy as jnp
from jax import lax
from jax.experimental import pallas as pl


def _knn_kernel(x_ref, xb_ref, f_ref, wcat_ref, pe_ref, out_ref, ploss_ref,
                *, R, K, J, CH):
    b = pl.program_id(0)
    i = pl.program_id(1)
    N = x_ref.shape[1]
    C = x_ref.shape[2]

    x_all = x_ref[0]
    fm = f_ref[0] > 0.1
    x_crd = jnp.where(fm, 0.0, x_all)
    x_ftr = jnp.where(fm, x_all, 0.0)

    n_f = jnp.float32(N)
    mean_c = jnp.sum(x_crd, axis=0, keepdims=True) / n_f
    mean_f = jnp.sum(x_ftr, axis=0, keepdims=True) / n_f
    var_c = jnp.sum((x_crd - mean_c) ** 2, axis=0, keepdims=True) / n_f
    var_f = jnp.sum((x_ftr - mean_f) ** 2, axis=0, keepdims=True) / n_f
    xn_c = jnp.clip((x_crd - mean_c) / (jnp.sqrt(var_c) + 1e-5), -10.0, 10.0)
    xn_f = jnp.clip((x_ftr - mean_f) / (jnp.sqrt(var_f) + 1e-5), -10.0, 10.0)

    sq_col = jnp.sum(x_crd * x_crd, axis=1, keepdims=True)
    tcol_c = jnp.sum(xn_c, axis=1, keepdims=True)
    tcol_f = jnp.sum(xn_f, axis=1, keepdims=True)

    x_blk_raw = xb_ref[0]
    x_blk = jnp.where(fm, 0.0, x_blk_raw)
    x_blk_f = jnp.where(fm, x_blk_raw, 0.0)
    xnc_blk = jnp.clip((x_blk - mean_c) / (jnp.sqrt(var_c) + 1e-5), -10.0, 10.0)
    xnf_blk = jnp.clip((x_blk_f - mean_f) / (jnp.sqrt(var_f) + 1e-5), -10.0, 10.0)
    tc_blk = jnp.sum(xnc_blk, axis=1, keepdims=True)
    tf_blk = jnp.sum(xnf_blk, axis=1, keepdims=True)

    ones_c = jnp.ones((1, C), jnp.float32)
    sqb_row = lax.dot_general(
        ones_c, x_blk * x_blk, (((1,), (1,)), ((), ())),
        preferred_element_type=jnp.float32,
        precision=lax.Precision.HIGHEST)

    g_t = lax.dot_general(
        x_crd, x_blk, (((1,), (1,)), ((), ())),
        preferred_element_type=jnp.float32,
        precision=lax.Precision.DEFAULT)
    d2t = sq_col + sqb_row - 2.0 * g_t
    dmt = jnp.sqrt(jnp.maximum(d2t, 0.0))

    SL = N // CH
    d3 = dmt.reshape(SL, CH, R)
    tc3 = tcol_c.reshape(SL, CH, 1)
    tf3 = tcol_f.reshape(SL, CH, 1)
    l3 = lax.broadcasted_iota(jnp.int32, (SL, CH, R), 0)
    big_l = jnp.int32(SL)
    ci2d = lax.broadcasted_iota(jnp.int32, (CH, R), 0)
    vals, gidxs, tcs1, tfs1 = [], [], [], []
    for _ in range(J):
        m = jnp.min(d3, axis=0, keepdims=True)
        cand = jnp.where(d3 == m, l3, big_l)
        li = jnp.min(cand, axis=0, keepdims=True)
        sel = cand == li
        vals.append(m[0])
        gidxs.append(li[0] * CH + ci2d)
        tcs1.append(jnp.sum(jnp.where(sel, tc3, 0.0), axis=0))
        tfs1.append(jnp.sum(jnp.where(sel, tf3, 0.0), axis=0))
        d3 = jnp.where(sel, jnp.inf, d3)

    v_sl = jnp.concatenate(vals, axis=0)
    g_sl = jnp.concatenate(gidxs, axis=0)
    tc_sl = jnp.concatenate(tcs1, axis=0)
    tf_sl = jnp.concatenate(tfs1, axis=0)

    big_g = jnp.int32(N)
    tc_out, tf_out = [], []
    for _ in range(K):
        m2 = jnp.min(v_sl, axis=0, keepdims=True)
        cand2 = jnp.where(v_sl == m2, g_sl, big_g)
        g2 = jnp.min(cand2, axis=0, keepdims=True)
        sel2 = cand2 == g2
        tc_out.append(jnp.sum(jnp.where(sel2, tc_sl, 0.0), axis=0, keepdims=True))
        tf_out.append(jnp.sum(jnp.where(sel2, tf_sl, 0.0), axis=0, keepdims=True))
        v_sl = jnp.where(sel2, jnp.inf, v_sl)

    s_t = jnp.concatenate(tc_out + tf_out, axis=0)
    wcat = wcat_ref[...]
    out = lax.dot_general(
        s_t, wcat, (((0,), (0,)), ((), ())),
        preferred_element_type=jnp.float32,
        precision=lax.Precision.HIGHEST)
    wc_sum = jnp.sum(wcat[:K, :], axis=0, keepdims=True)
    wf_sum = jnp.sum(wcat[K:, :], axis=0, keepdims=True)
    pe = pe_ref[...]
    pe_sum = jnp.sum(pe, axis=0, keepdims=True)
    out_ref[0] = out - tc_blk * wc_sum - tf_blk * wf_sum + pe_sum

    @pl.when((b == 0) & (i == 0))
    def _():
        ploss_ref[...] = jnp.sum(jnp.abs(pe), keepdims=True)


def kernel(x, features, attn_mask, W_crd, W_ftr, pe_crd, pe_ftr):
    del attn_mask
    B, N, C = x.shape
    D, K = W_crd.shape
    R = 512 if N % 512 == 0 else N
    CH = 16 if N % 16 == 0 else 1
    J = 8 if CH > 1 else min(K, N)

    wcat = jnp.concatenate([W_crd.T, W_ftr.T], axis=0)
    pe_cat = jnp.concatenate(
        [pe_crd.reshape(C, D), pe_ftr.reshape(C, D)], axis=0
    )
    f3 = features.reshape(B, 1, C)

    out, ploss = pl.pallas_call(
        functools.partial(_knn_kernel, R=R, K=K, J=J, CH=CH),
        grid=(B, N // R),
        in_specs=[
            pl.BlockSpec((1, N, C), lambda b, i: (b, 0, 0)),
            pl.BlockSpec((1, R, C), lambda b, i: (b, i, 0)),
            pl.BlockSpec((1, 1, C), lambda b, i: (b, 0, 0)),
            pl.BlockSpec((2 * K, D), lambda b, i: (0, 0)),
            pl.BlockSpec((2 * C, D), lambda b, i: (0, 0)),
        ],
        out_specs=[
            pl.BlockSpec((1, R, D), lambda b, i: (b, i, 0)),
            pl.BlockSpec((1, 1), lambda b, i: (0, 0)),
        ],
        out_shape=[
            jax.ShapeDtypeStruct((B, N, D), jnp.float32),
            jax.ShapeDtypeStruct((1, 1), jnp.float32),
        ],
    )(x, x, f3, wcat, pe_cat)
    return out, ploss.reshape(())
